# baseline (speedup 1.0000x reference)
import functools

import jax
import jax.numpy as jnp
import numpy as np
from jax.experimental import pallas as pl
from jax.experimental.pallas import tpu as pltpu

NEG_SLOPE = 0.01
BN_EPS = 1e-5


@functools.lru_cache(maxsize=None)
def _sel_matrix(Wo, Wp, LO, SEL_K):
    t = np.arange(LO)
    src = (t // Wo) * 2 * Wp + 2 * (t % Wo)
    m = np.zeros((SEL_K, LO), dtype=np.float32)
    m[src, t] = 1.0
    return m.astype(jnp.bfloat16)


def kernel(x, weight, bias, gamma, beta):
    del bias
    N, Cin, H, W = x.shape
    Cout, Cin2, K, K2 = weight.shape
    assert Cin2 == Cin and K == K2 == 3
    pad = K // 2
    assert H % 2 == 0 and W % 2 == 0
    Ho, Wo = H // 2, W // 2
    Hp, Wp = H + 2 * pad, W + 2 * pad
    FL = Hp * Wp
    CONV_L = (H - 1) * Wp + W
    LO = Ho * Wo
    KKC = K * K * Cin
    inv_count = 1.0 / float(N * Ho * Wo)

    xb = x.astype(jnp.bfloat16)
    xp = jnp.pad(xb, ((0, 0), (0, 0), (pad, pad), (pad, pad)), mode="reflect")
    xf = xp.reshape(N, Cin, FL)

    wmat = weight.transpose(0, 2, 3, 1).reshape(Cout, KKC).astype(jnp.bfloat16)

    SEL_K = 2 * (Ho - 1) * Wp + 2 * (Wo - 1) + 1
    sel = jnp.asarray(_sel_matrix(Wo, Wp, LO, SEL_K))

    def conv_pool_stats_kernel(x_ref, w_ref, sel_ref, pooled_ref, stats_ref):
        xa = x_ref[0]
        w = w_ref[...]
        selm = sel_ref[...]

        pieces = [
            xa[:, kh * Wp + kw:kh * Wp + kw + CONV_L]
            for kh in range(K) for kw in range(K)
        ]
        patch = jnp.concatenate(pieces, axis=0)
        conv = jnp.dot(w, patch, preferred_element_type=jnp.float32)

        hmax = jnp.maximum(conv[:, :CONV_L - Wp], conv[:, Wp:])
        wmax = jnp.maximum(hmax[:, :SEL_K], hmax[:, 1:SEL_K + 1])
        pooled = jnp.dot(wmax.astype(jnp.bfloat16), selm,
                         preferred_element_type=jnp.float32)
        pooled_ref[0] = pooled.astype(jnp.bfloat16)

        s1 = jnp.sum(pooled, axis=1, keepdims=True)
        s2 = jnp.sum(pooled * pooled, axis=1, keepdims=True)
        stats_ref[0] = jnp.concatenate([s1, s2], axis=1)

    pooled_parts, stats_parts = pl.pallas_call(
        conv_pool_stats_kernel,
        out_shape=(
            jax.ShapeDtypeStruct((N, Cout, LO), jnp.bfloat16),
            jax.ShapeDtypeStruct((N, Cout, 2), jnp.float32),
        ),
        grid=(N,),
        in_specs=[
            pl.BlockSpec((1, Cin, FL), lambda n: (n, 0, 0)),
            pl.BlockSpec((Cout, KKC), lambda n: (0, 0)),
            pl.BlockSpec((SEL_K, LO), lambda n: (0, 0)),
        ],
        out_specs=(
            pl.BlockSpec((1, Cout, LO), lambda n: (n, 0, 0)),
            pl.BlockSpec((1, Cout, 2), lambda n: (n, 0, 0)),
        ),
        compiler_params=pltpu.CompilerParams(dimension_semantics=("parallel",)),
    )(xf, wmat, sel)

    stats_tot = jnp.sum(stats_parts, axis=0)
    params = jnp.concatenate(
        [gamma.reshape(Cout, 1), beta.reshape(Cout, 1), stats_tot], axis=1
    ).astype(jnp.float32)

    def bn_act_kernel(pooled_ref, params_ref, out_ref):
        po = pooled_ref[0].astype(jnp.float32)
        prm = params_ref[...]
        gam, bet = prm[:, 0:1], prm[:, 1:2]
        mean = prm[:, 2:3] * inv_count
        var = prm[:, 3:4] * inv_count - mean * mean
        var = jnp.maximum(var, 0.0)
        scale = gam * jax.lax.rsqrt(var + BN_EPS)
        shift = bet - mean * scale
        z = po * scale + shift
        out_ref[0] = jnp.maximum(z, NEG_SLOPE * z)

    y_parts = pl.pallas_call(
        bn_act_kernel,
        out_shape=jax.ShapeDtypeStruct((N, Cout, LO), jnp.float32),
        grid=(N,),
        in_specs=[
            pl.BlockSpec((1, Cout, LO), lambda n: (n, 0, 0)),
            pl.BlockSpec((Cout, 4), lambda n: (0, 0)),
        ],
        out_specs=pl.BlockSpec((1, Cout, LO), lambda n: (n, 0, 0)),
        compiler_params=pltpu.CompilerParams(dimension_semantics=("parallel",)),
    )(pooled_parts, params)

    return y_parts.reshape(N, Cout, Ho, Wo)


# speedup vs baseline: 1.1529x; 1.1529x over previous
import functools

import jax
import jax.numpy as jnp
import numpy as np
from jax.experimental import pallas as pl
from jax.experimental.pallas import tpu as pltpu

NEG_SLOPE = 0.01
BN_EPS = 1e-5


@functools.lru_cache(maxsize=None)
def _sel_matrix(Wo, Wp, LO, SEL_K):
    t = np.arange(LO)
    src = (t // Wo) * 2 * Wp + 2 * (t % Wo)
    m = np.zeros((SEL_K, LO), dtype=np.float32)
    m[src, t] = 1.0
    return m.astype(jnp.bfloat16)


def kernel(x, weight, bias, gamma, beta):
    del bias
    N, Cin, H, W = x.shape
    Cout, Cin2, K, K2 = weight.shape
    assert Cin2 == Cin and K == K2 == 3
    pad = K // 2
    assert H % 2 == 0 and W % 2 == 0
    Ho, Wo = H // 2, W // 2
    Hp, Wp = H + 2 * pad, W + 2 * pad
    FL = Hp * Wp
    CONV_L = (H - 1) * Wp + W
    LO = Ho * Wo
    KKC = K * K * Cin
    inv_count = 1.0 / float(N * Ho * Wo)

    xp = jnp.pad(x, ((0, 0), (0, 0), (pad, pad), (pad, pad)), mode="reflect")
    xf = xp.reshape(N, Cin, FL).astype(jnp.bfloat16)

    wmat = weight.transpose(0, 2, 3, 1).reshape(Cout, KKC).astype(jnp.bfloat16)

    SEL_K = 2 * (Ho - 1) * Wp + 2 * (Wo - 1) + 1
    sel = jnp.asarray(_sel_matrix(Wo, Wp, LO, SEL_K))

    def conv_pool_stats_kernel(x_ref, w_ref, sel_ref, pooled_ref, stats_ref):
        xa = x_ref[0]
        w = w_ref[...]
        selm = sel_ref[...]

        pieces = [
            xa[:, kh * Wp + kw:kh * Wp + kw + CONV_L]
            for kh in range(K) for kw in range(K)
        ]
        patch = jnp.concatenate(pieces, axis=0)
        conv = jnp.dot(w, patch, preferred_element_type=jnp.float32)

        hmax = jnp.maximum(conv[:, :CONV_L - Wp], conv[:, Wp:])
        wmax = jnp.maximum(hmax[:, :SEL_K], hmax[:, 1:SEL_K + 1])
        pooled = jnp.dot(wmax.astype(jnp.bfloat16), selm,
                         preferred_element_type=jnp.float32)
        pooled_ref[0] = pooled.astype(jnp.bfloat16)

        s1 = jnp.sum(pooled, axis=1, keepdims=True)
        s2 = jnp.sum(pooled * pooled, axis=1, keepdims=True)
        stats_ref[0] = jnp.concatenate([s1, s2], axis=1)

    pooled_parts, stats_parts = pl.pallas_call(
        conv_pool_stats_kernel,
        out_shape=(
            jax.ShapeDtypeStruct((N, Cout, LO), jnp.bfloat16),
            jax.ShapeDtypeStruct((N, Cout, 2), jnp.float32),
        ),
        grid=(N,),
        in_specs=[
            pl.BlockSpec((1, Cin, FL), lambda n: (n, 0, 0)),
            pl.BlockSpec((Cout, KKC), lambda n: (0, 0)),
            pl.BlockSpec((SEL_K, LO), lambda n: (0, 0)),
        ],
        out_specs=(
            pl.BlockSpec((1, Cout, LO), lambda n: (n, 0, 0)),
            pl.BlockSpec((1, Cout, 2), lambda n: (n, 0, 0)),
        ),
        compiler_params=pltpu.CompilerParams(dimension_semantics=("parallel",)),
    )(xf, wmat, sel)

    stats_tot = jnp.sum(stats_parts, axis=0)
    params = jnp.concatenate(
        [gamma.reshape(Cout, 1), beta.reshape(Cout, 1), stats_tot], axis=1
    ).astype(jnp.float32)

    def bn_act_kernel(pooled_ref, params_ref, out_ref):
        po = pooled_ref[0].astype(jnp.float32)
        prm = params_ref[...]
        gam, bet = prm[:, 0:1], prm[:, 1:2]
        mean = prm[:, 2:3] * inv_count
        var = prm[:, 3:4] * inv_count - mean * mean
        var = jnp.maximum(var, 0.0)
        scale = gam * jax.lax.rsqrt(var + BN_EPS)
        shift = bet - mean * scale
        z = po * scale + shift
        out_ref[0] = jnp.maximum(z, NEG_SLOPE * z)

    y_parts = pl.pallas_call(
        bn_act_kernel,
        out_shape=jax.ShapeDtypeStruct((N, Cout, LO), jnp.float32),
        grid=(N,),
        in_specs=[
            pl.BlockSpec((1, Cout, LO), lambda n: (n, 0, 0)),
            pl.BlockSpec((Cout, 4), lambda n: (0, 0)),
        ],
        out_specs=pl.BlockSpec((1, Cout, LO), lambda n: (n, 0, 0)),
        compiler_params=pltpu.CompilerParams(dimension_semantics=("parallel",)),
    )(pooled_parts, params)

    return y_parts.reshape(N, Cout, Ho, Wo)
